# baseline (speedup 1.0000x reference)
import functools

import jax
import jax.numpy as jnp
from jax import lax
from jax.experimental import pallas as pl
from jax.experimental.pallas import tpu as pltpu
from jax.experimental.pallas import tpu_sc as plsc

N_NODES = 10000
N_EDGES = 320000
D = 128
D_EDGE = 16
N_GRAPHS = 16

NC = 2
NS = 16
NW = NC * NS
L = 16

E_PER_TILE = N_EDGES // NW
K = 40
CHUNKS = E_PER_TILE // K
RPT = 600
TAIL_ROWS = N_NODES - NS * RPT
ZR = 40

_HIGHEST = jax.lax.Precision.HIGHEST


def _matmul(a, b):
    return jnp.dot(a, b, preferred_element_type=jnp.float32, precision=_HIGHEST)


def _proj_body(x_ref, wd_ref, ws_ref, pd_ref, ps_ref):
    xv = x_ref[...]
    pd_ref[...] = _matmul(xv, wd_ref[...])
    ps_ref[...] = _matmul(xv, ws_ref[...])


def _node_proj(x, WD, WS):
    return pl.pallas_call(
        _proj_body,
        out_shape=[
            jax.ShapeDtypeStruct((N_NODES, 2 * D), jnp.float32),
            jax.ShapeDtypeStruct((N_NODES, 2 * D), jnp.float32),
        ],
    )(x, WD, WS)


E_BLK = 4000


def _eproj_body(ea_ref, we_ref, be_ref, e_ref):
    e_ref[...] = _matmul(ea_ref[...], we_ref[...]) + be_ref[...]


def _edge_proj(edge_attr, WE, bE):
    n_blk = N_EDGES // E_BLK
    return pl.pallas_call(
        _eproj_body,
        grid=(n_blk,),
        in_specs=[
            pl.BlockSpec((E_BLK, D_EDGE), lambda i: (i, 0)),
            pl.BlockSpec((D_EDGE, 2 * D), lambda i: (0, 0)),
            pl.BlockSpec((1, 2 * D), lambda i: (0, 0)),
        ],
        out_specs=pl.BlockSpec((E_BLK, 2 * D), lambda i: (i, 0)),
        out_shape=jax.ShapeDtypeStruct((N_EDGES, 2 * D), jnp.float32),
    )(edge_attr, WE, bE)


def _sigmoid(v):
    return 1.0 / (1.0 + jnp.exp(-v))


def _softplus(v):
    u = jnp.exp(-jnp.abs(v))
    s = u / (2.0 + u)
    s2 = s * s
    p = 1.0 + s2 * (0.33333334 + s2 * (0.2 + s2 * (0.14285715 + s2 * 0.11111111)))
    return jnp.maximum(v, 0.0) + 2.0 * s * p


def _edge_body(pd_hbm, ps_hbm, e_hbm, dst_hbm, src_hbm, out_hbm,
               dsti, srci, drows, srows, erows, msg, zbuf, acc,
               sem_d, sem_s, sem_e):
    c = lax.axis_index("c")
    s = lax.axis_index("s")
    wid = c * NS + s
    tile_base = wid * E_PER_TILE

    @pl.loop(0, ZR)
    def _zero(i):
        for j in range(D // L):
            zbuf[i, pl.ds(j * L, L)] = jnp.zeros((L,), jnp.float32)

    @pl.loop(0, RPT // ZR)
    def _zcp(r):
        pltpu.sync_copy(zbuf, acc.at[pl.ds(s * RPT + r * ZR, ZR)])

    @pl.when(s == NS - 1)
    def _zero_tail():
        @pl.loop(0, TAIL_ROWS // ZR)
        def _ztail(r):
            pltpu.sync_copy(zbuf, acc.at[pl.ds(NS * RPT + r * ZR, ZR)])

    plsc.subcore_barrier()

    @pl.loop(0, CHUNKS)
    def _chunk(ci):
        base = tile_base + ci * K
        pltpu.sync_copy(dst_hbm.at[pl.ds(base, K)], dsti)
        pltpu.sync_copy(src_hbm.at[pl.ds(base, K)], srci)
        cp_d = pltpu.async_copy(pd_hbm.at[dsti], drows, sem_d)
        cp_s = pltpu.async_copy(ps_hbm.at[srci], srows, sem_s)
        cp_e = pltpu.async_copy(e_hbm.at[pl.ds(base, K), :], erows, sem_e)
        cp_d.wait()
        cp_s.wait()
        cp_e.wait()

        @pl.loop(0, K)
        def _edge(i):
            for j in range(D // L):
                o = j * L
                zf = (drows[i, pl.ds(o, L)] + srows[i, pl.ds(o, L)]
                      + erows[i, pl.ds(o, L)])
                zs = (drows[i, pl.ds(D + o, L)] + srows[i, pl.ds(D + o, L)]
                      + erows[i, pl.ds(D + o, L)])
                msg[i, pl.ds(o, L)] = _sigmoid(zf) * _softplus(zs)

        pltpu.sync_copy(msg, acc.at[dsti], add=True)

    plsc.subcore_barrier()
    pltpu.sync_copy(acc.at[pl.ds(s * RPT, RPT)],
                    out_hbm.at[c, pl.ds(s * RPT, RPT)])

    @pl.when(s == NS - 1)
    def _drain_tail():
        pltpu.sync_copy(acc.at[pl.ds(NS * RPT, TAIL_ROWS)],
                        out_hbm.at[c, pl.ds(NS * RPT, TAIL_ROWS)])


def _edge_stage(pd, ps, e, dst, src):
    mesh = plsc.VectorSubcoreMesh(core_axis_name="c", subcore_axis_name="s")
    run = pl.kernel(
        _edge_body,
        out_type=jax.ShapeDtypeStruct((NC, N_NODES, D), jnp.float32),
        mesh=mesh,
        scratch_types=[
            pltpu.VMEM((K,), jnp.int32),
            pltpu.VMEM((K,), jnp.int32),
            pltpu.VMEM((K, 2 * D), jnp.float32),
            pltpu.VMEM((K, 2 * D), jnp.float32),
            pltpu.VMEM((K, 2 * D), jnp.float32),
            pltpu.VMEM((K, D), jnp.float32),
            pltpu.VMEM((ZR, D), jnp.float32),
            pltpu.VMEM_SHARED((N_NODES, D), jnp.float32),
            pltpu.SemaphoreType.DMA,
            pltpu.SemaphoreType.DMA,
            pltpu.SemaphoreType.DMA,
        ],
    )
    return run(pd, ps, e, dst, src)


def _tail_body(x_ref, acc_ref, batch_ref, w1_ref, b1_ref, w2_ref, b2_ref, out_ref):
    h = jax.nn.sigmoid(x_ref[...] + acc_ref[0] + acc_ref[1])
    h = jax.nn.sigmoid(_matmul(h, w1_ref[...]) + b1_ref[...])
    gids = lax.broadcasted_iota(jnp.int32, (N_GRAPHS, N_NODES), 0)
    onehot = (gids == batch_ref[...]).astype(jnp.float32)
    pooled = _matmul(onehot, h)
    out_ref[...] = jnp.maximum(_matmul(pooled, w2_ref[...]) + b2_ref[...], 0.0)


def _tail(x, acc, batch2d, W1, b1, W2, b2):
    return pl.pallas_call(
        _tail_body,
        out_shape=jax.ShapeDtypeStruct((N_GRAPHS, 1), jnp.float32),
    )(x, acc, batch2d, W1, b1, W2, b2)


def kernel(x, edge_index, edge_attr, batch, W_f, b_f, W_s, b_s, W1, b1, W2, b2):
    src = edge_index[0]
    dst = edge_index[1]
    WD = jnp.concatenate([W_f[:D], W_s[:D]], axis=1)
    WS = jnp.concatenate([W_f[D:2 * D], W_s[D:2 * D]], axis=1)
    WE = jnp.concatenate([W_f[2 * D:], W_s[2 * D:]], axis=1)
    bE = jnp.concatenate([b_f, b_s]).reshape(1, 2 * D)

    pd, ps = _node_proj(x, WD, WS)
    e = _edge_proj(edge_attr, WE, bE)
    acc = _edge_stage(pd, ps, e, dst, src)
    return _tail(x, acc, batch.reshape(1, N_NODES),
                 W1, b1.reshape(1, 6), W2, b2.reshape(1, 1))


# speedup vs baseline: 1.2240x; 1.2240x over previous
import functools

import jax
import jax.numpy as jnp
from jax import lax
from jax.experimental import pallas as pl
from jax.experimental.pallas import tpu as pltpu
from jax.experimental.pallas import tpu_sc as plsc

N_NODES = 10000
N_EDGES = 320000
D = 128
D_EDGE = 16
N_GRAPHS = 16

NC = 2
NS = 16
NW = NC * NS
L = 16

E_PER_TILE = N_EDGES // NW
K = 40
CHUNKS = E_PER_TILE // K
RPT = 600
TAIL_ROWS = N_NODES - NS * RPT
ZR = 40

_HIGHEST = jax.lax.Precision.HIGHEST


def _matmul(a, b):
    return jnp.dot(a, b, preferred_element_type=jnp.float32, precision=_HIGHEST)


def _proj_body(x_ref, wd_ref, ws_ref, pd_ref, ps_ref):
    xv = x_ref[...]
    pd_ref[...] = _matmul(xv, wd_ref[...])
    ps_ref[...] = _matmul(xv, ws_ref[...])


def _node_proj(x, WD, WS):
    return pl.pallas_call(
        _proj_body,
        out_shape=[
            jax.ShapeDtypeStruct((N_NODES, 2 * D), jnp.float32),
            jax.ShapeDtypeStruct((N_NODES, 2 * D), jnp.float32),
        ],
    )(x, WD, WS)


E_BLK = 4000


def _eproj_body(ea_ref, we_ref, be_ref, e_ref):
    e_ref[...] = _matmul(ea_ref[...], we_ref[...]) + be_ref[...]


def _edge_proj(edge_attr, WE, bE):
    n_blk = N_EDGES // E_BLK
    return pl.pallas_call(
        _eproj_body,
        grid=(n_blk,),
        in_specs=[
            pl.BlockSpec((E_BLK, D_EDGE), lambda i: (i, 0)),
            pl.BlockSpec((D_EDGE, 2 * D), lambda i: (0, 0)),
            pl.BlockSpec((1, 2 * D), lambda i: (0, 0)),
        ],
        out_specs=pl.BlockSpec((E_BLK, 2 * D), lambda i: (i, 0)),
        out_shape=jax.ShapeDtypeStruct((N_EDGES, 2 * D), jnp.float32),
    )(edge_attr, WE, bE)


def _sigmoid(v):
    return 1.0 / (1.0 + jnp.exp(-v))


def _softplus(v):
    u = jnp.exp(-jnp.abs(v))
    s = u / (2.0 + u)
    s2 = s * s
    p = 1.0 + s2 * (0.33333334 + s2 * (0.2 + s2 * (0.14285715 + s2 * 0.11111111)))
    return jnp.maximum(v, 0.0) + 2.0 * s * p


def _edge_body(pd_hbm, ps_hbm, e_hbm, dst_hbm, src_hbm, out_hbm,
               dsti, srci, drows, srows, erows, msg, zbuf, acc,
               sem_d, sem_s, sem_e):
    c = lax.axis_index("c")
    s = lax.axis_index("s")
    wid = c * NS + s
    tile_base = wid * E_PER_TILE

    @pl.loop(0, ZR)
    def _zero(i):
        for j in range(D // L):
            zbuf[i, pl.ds(j * L, L)] = jnp.zeros((L,), jnp.float32)

    @pl.loop(0, RPT // ZR)
    def _zcp(r):
        pltpu.sync_copy(zbuf, acc.at[pl.ds(s * RPT + r * ZR, ZR)])

    @pl.when(s == NS - 1)
    def _zero_tail():
        @pl.loop(0, TAIL_ROWS // ZR)
        def _ztail(r):
            pltpu.sync_copy(zbuf, acc.at[pl.ds(NS * RPT + r * ZR, ZR)])

    plsc.subcore_barrier()

    @pl.loop(0, CHUNKS)
    def _chunk(ci):
        base = tile_base + ci * K
        pltpu.sync_copy(dst_hbm.at[pl.ds(base, K)], dsti)
        pltpu.sync_copy(src_hbm.at[pl.ds(base, K)], srci)
        cp_d = pltpu.async_copy(pd_hbm.at[dsti], drows, sem_d)
        cp_s = pltpu.async_copy(ps_hbm.at[srci], srows, sem_s)
        cp_e = pltpu.async_copy(e_hbm.at[pl.ds(base, K), :], erows, sem_e)
        cp_d.wait()
        cp_s.wait()
        cp_e.wait()

        @plsc.parallel_loop(0, K, unroll=4)
        def _edge(i):
            for j in range(D // L):
                o = j * L
                zf = (drows[i, pl.ds(o, L)] + srows[i, pl.ds(o, L)]
                      + erows[i, pl.ds(o, L)])
                zs = (drows[i, pl.ds(D + o, L)] + srows[i, pl.ds(D + o, L)]
                      + erows[i, pl.ds(D + o, L)])
                den = 1.0 + jnp.exp(-zf)
                u = jnp.exp(jnp.minimum(zs, -zs))
                sarg = u / (2.0 + u)
                s2 = sarg * sarg
                p = 1.0 + s2 * (0.33333334 + s2 * (0.2 + s2 * 0.14285715))
                num = jnp.maximum(zs, 0.0) + 2.0 * sarg * p
                msg[i, pl.ds(o, L)] = num / den

        pltpu.sync_copy(msg, acc.at[dsti], add=True)

    plsc.subcore_barrier()
    pltpu.sync_copy(acc.at[pl.ds(s * RPT, RPT)],
                    out_hbm.at[c, pl.ds(s * RPT, RPT)])

    @pl.when(s == NS - 1)
    def _drain_tail():
        pltpu.sync_copy(acc.at[pl.ds(NS * RPT, TAIL_ROWS)],
                        out_hbm.at[c, pl.ds(NS * RPT, TAIL_ROWS)])


def _edge_stage(pd, ps, e, dst, src):
    mesh = plsc.VectorSubcoreMesh(core_axis_name="c", subcore_axis_name="s")
    run = pl.kernel(
        _edge_body,
        out_type=jax.ShapeDtypeStruct((NC, N_NODES, D), jnp.float32),
        mesh=mesh,
        scratch_types=[
            pltpu.VMEM((K,), jnp.int32),
            pltpu.VMEM((K,), jnp.int32),
            pltpu.VMEM((K, 2 * D), jnp.float32),
            pltpu.VMEM((K, 2 * D), jnp.float32),
            pltpu.VMEM((K, 2 * D), jnp.float32),
            pltpu.VMEM((K, D), jnp.float32),
            pltpu.VMEM((ZR, D), jnp.float32),
            pltpu.VMEM_SHARED((N_NODES, D), jnp.float32),
            pltpu.SemaphoreType.DMA,
            pltpu.SemaphoreType.DMA,
            pltpu.SemaphoreType.DMA,
        ],
    )
    return run(pd, ps, e, dst, src)


def _tail_body(x_ref, acc_ref, batch_ref, w1_ref, b1_ref, w2_ref, b2_ref, out_ref):
    h = jax.nn.sigmoid(x_ref[...] + acc_ref[0] + acc_ref[1])
    h = jax.nn.sigmoid(_matmul(h, w1_ref[...]) + b1_ref[...])
    gids = lax.broadcasted_iota(jnp.int32, (N_GRAPHS, N_NODES), 0)
    onehot = (gids == batch_ref[...]).astype(jnp.float32)
    pooled = _matmul(onehot, h)
    out_ref[...] = jnp.maximum(_matmul(pooled, w2_ref[...]) + b2_ref[...], 0.0)


def _tail(x, acc, batch2d, W1, b1, W2, b2):
    return pl.pallas_call(
        _tail_body,
        out_shape=jax.ShapeDtypeStruct((N_GRAPHS, 1), jnp.float32),
    )(x, acc, batch2d, W1, b1, W2, b2)


def kernel(x, edge_index, edge_attr, batch, W_f, b_f, W_s, b_s, W1, b1, W2, b2):
    src = edge_index[0]
    dst = edge_index[1]
    WD = jnp.concatenate([W_f[:D], W_s[:D]], axis=1)
    WS = jnp.concatenate([W_f[D:2 * D], W_s[D:2 * D]], axis=1)
    WE = jnp.concatenate([W_f[2 * D:], W_s[2 * D:]], axis=1)
    bE = jnp.concatenate([b_f, b_s]).reshape(1, 2 * D)

    pd, ps = _node_proj(x, WD, WS)
    e = _edge_proj(edge_attr, WE, bE)
    acc = _edge_stage(pd, ps, e, dst, src)
    return _tail(x, acc, batch.reshape(1, N_NODES),
                 W1, b1.reshape(1, 6), W2, b2.reshape(1, 1))
